# baseline (speedup 1.0000x reference)
import jax
import jax.numpy as jnp
from jax import lax
from jax.experimental import pallas as pl
from jax.experimental.pallas import tpu as pltpu
from jax.experimental.pallas import tpu_sc as plsc

MAX_LEN = 4096
EMB = 128
BATCH = 4

_info = plsc.get_sparse_core_info()
_NC, _NS, _L = _info.num_cores, _info.num_subcores, _info.num_lanes
_NW = _NC * _NS
_CHUNK = (MAX_LEN // _NW) * EMB
_VECS = _CHUNK // _L
_UNROLL = 8
_SEG = 4
_SEG_VECS = _VECS // _SEG
_SEG_ELEMS = _SEG_VECS * _L


def _tpe_body(x_hbm, pos_hbm, out_hbm, pos_v, xb_v, sem_pos, sem_x, *store_sems):
    wid = lax.axis_index("s") * _NC + lax.axis_index("c")
    base = wid * _CHUNK

    pos_copy = pltpu.async_copy(pos_hbm.at[pl.ds(base, _CHUNK)], pos_v, sem_pos)
    x_copy = pltpu.async_copy(x_hbm.at[:, pl.ds(base, _CHUNK)], xb_v, sem_x)
    pos_copy.wait()
    x_copy.wait()

    stores = []
    for s in range(_SEG):

        @plsc.parallel_loop(0, _SEG_VECS, step=1, unroll=_UNROLL)
        def add_body(i, s=s):
            sl = pl.ds(s * _SEG_ELEMS + i * _L, _L)
            p = pos_v[sl]
            for b in range(BATCH):
                xb_v[b, sl] = xb_v[b, sl] + p

        stores.append(pltpu.async_copy(
            xb_v.at[:, pl.ds(s * _SEG_ELEMS, _SEG_ELEMS)],
            out_hbm.at[:, pl.ds(base + s * _SEG_ELEMS, _SEG_ELEMS)],
            store_sems[s]))
    for st in stores:
        st.wait()


def kernel(x, pos_table):
    x2d = x.reshape(BATCH, MAX_LEN * EMB)
    pos_flat = pos_table.reshape(-1)
    mesh = plsc.VectorSubcoreMesh(core_axis_name="c", subcore_axis_name="s")
    scratch = [
        pltpu.VMEM((_CHUNK,), jnp.float32),
        pltpu.VMEM((BATCH, _CHUNK), jnp.float32),
    ] + [pltpu.SemaphoreType.DMA] * (2 + _SEG)
    out = pl.kernel(
        _tpe_body,
        mesh=mesh,
        out_type=jax.ShapeDtypeStruct((BATCH, MAX_LEN * EMB), jnp.float32),
        scratch_types=scratch,
    )(x2d, pos_flat)
    return out.reshape(BATCH, MAX_LEN, EMB)


# speedup vs baseline: 1.6218x; 1.6218x over previous
import jax
import jax.numpy as jnp
from jax import lax
from jax.experimental import pallas as pl
from jax.experimental.pallas import tpu as pltpu
from jax.experimental.pallas import tpu_sc as plsc

MAX_LEN = 4096
EMB = 128
BATCH = 4

_info = plsc.get_sparse_core_info()
_NC, _NS, _L = _info.num_cores, _info.num_subcores, _info.num_lanes
_NW = _NC * _NS
_CHUNK = (MAX_LEN // _NW) * EMB
_VECS = _CHUNK // _L
_UNROLL = 16
_SEG = 2
_SEG_VECS = _VECS // _SEG
_SEG_ELEMS = _SEG_VECS * _L


def _tpe_body(x_hbm, pos_hbm, out_hbm, pos_v, xb_v, sem_pos, *sems):
    wid = lax.axis_index("s") * _NC + lax.axis_index("c")
    base = wid * _CHUNK
    load_sems = sems[:BATCH]
    store_sems = sems[BATCH:]

    pos_copy = pltpu.async_copy(pos_hbm.at[pl.ds(base, _CHUNK)], pos_v, sem_pos)
    loads = [
        pltpu.async_copy(
            x_hbm.at[pl.ds(b * (MAX_LEN * EMB) + base, _CHUNK)],
            xb_v.at[b], load_sems[b])
        for b in range(BATCH)
    ]
    pos_copy.wait()

    stores = []
    for b in range(BATCH):
        loads[b].wait()
        for s in range(_SEG):

            @plsc.parallel_loop(0, _SEG_VECS, step=1, unroll=_UNROLL)
            def add_body(i, b=b, s=s):
                sl = pl.ds(s * _SEG_ELEMS + i * _L, _L)
                xb_v[b, sl] = xb_v[b, sl] + pos_v[sl]

            stores.append(pltpu.async_copy(
                xb_v.at[b, pl.ds(s * _SEG_ELEMS, _SEG_ELEMS)],
                out_hbm.at[pl.ds(b * (MAX_LEN * EMB) + base + s * _SEG_ELEMS,
                                 _SEG_ELEMS)],
                store_sems[b]))
    for st in stores:
        st.wait()


def kernel(x, pos_table):
    x_flat = x.reshape(-1)
    pos_flat = pos_table.reshape(-1)
    mesh = plsc.VectorSubcoreMesh(core_axis_name="c", subcore_axis_name="s")
    scratch = [
        pltpu.VMEM((_CHUNK,), jnp.float32),
        pltpu.VMEM((BATCH, _CHUNK), jnp.float32),
    ] + [pltpu.SemaphoreType.DMA] * (1 + 2 * BATCH)
    out = pl.kernel(
        _tpe_body,
        mesh=mesh,
        out_type=jax.ShapeDtypeStruct((BATCH * MAX_LEN * EMB,), jnp.float32),
        scratch_types=scratch,
    )(x_flat, pos_flat)
    return out.reshape(BATCH, MAX_LEN, EMB)


# speedup vs baseline: 1.6669x; 1.0278x over previous
import jax
import jax.numpy as jnp
from jax import lax
from jax.experimental import pallas as pl
from jax.experimental.pallas import tpu as pltpu
from jax.experimental.pallas import tpu_sc as plsc

MAX_LEN = 4096
EMB = 128
BATCH = 4

_info = plsc.get_sparse_core_info()
_NC, _NS, _L = _info.num_cores, _info.num_subcores, _info.num_lanes
_NW = _NC * _NS
_CHUNK = (MAX_LEN // _NW) * EMB
_VECS = _CHUNK // _L
_UNROLL = 8
_SEG = 1
_SEG_VECS = _VECS // _SEG
_SEG_ELEMS = _SEG_VECS * _L


def _tpe_body(x_hbm, pos_hbm, out_hbm, pos_v, xb_v, sem_pos, *sems):
    wid = lax.axis_index("s") * _NC + lax.axis_index("c")
    base = wid * _CHUNK
    load_sems = sems[:BATCH]
    store_sems = sems[BATCH:]

    pos_copy = pltpu.async_copy(pos_hbm.at[pl.ds(base, _CHUNK)], pos_v, sem_pos)
    loads = [
        pltpu.async_copy(
            x_hbm.at[pl.ds(b * (MAX_LEN * EMB) + base, _CHUNK)],
            xb_v.at[b], load_sems[b])
        for b in range(BATCH)
    ]
    pos_copy.wait()

    stores = []
    for b in range(BATCH):
        loads[b].wait()
        for s in range(_SEG):

            @plsc.parallel_loop(0, _SEG_VECS, step=1, unroll=_UNROLL)
            def add_body(i, b=b, s=s):
                sl = pl.ds(s * _SEG_ELEMS + i * _L, _L)
                xb_v[b, sl] = xb_v[b, sl] + pos_v[sl]

            stores.append(pltpu.async_copy(
                xb_v.at[b, pl.ds(s * _SEG_ELEMS, _SEG_ELEMS)],
                out_hbm.at[pl.ds(b * (MAX_LEN * EMB) + base + s * _SEG_ELEMS,
                                 _SEG_ELEMS)],
                store_sems[b]))
    for st in stores:
        st.wait()


def kernel(x, pos_table):
    x_flat = x.reshape(-1)
    pos_flat = pos_table.reshape(-1)
    mesh = plsc.VectorSubcoreMesh(core_axis_name="c", subcore_axis_name="s")
    scratch = [
        pltpu.VMEM((_CHUNK,), jnp.float32),
        pltpu.VMEM((BATCH, _CHUNK), jnp.float32),
    ] + [pltpu.SemaphoreType.DMA] * (1 + 2 * BATCH)
    out = pl.kernel(
        _tpe_body,
        mesh=mesh,
        out_type=jax.ShapeDtypeStruct((BATCH * MAX_LEN * EMB,), jnp.float32),
        scratch_types=scratch,
    )(x_flat, pos_flat)
    return out.reshape(BATCH, MAX_LEN, EMB)
